# baseline (speedup 1.0000x reference)
import functools

import jax
import jax.numpy as jnp
from jax.experimental import pallas as pl
from jax.experimental.pallas import tpu as pltpu

_BN_EPS = 1e-5
_VMEM_LIMIT = 64 * 1024 * 1024
_IPB = 4


def _prep_kernel(ipb, x_ref, xp_ref):
    for j in range(ipb):
        xt = jnp.transpose(x_ref[j].astype(jnp.bfloat16), (1, 2, 0))
        xp_ref[j] = jnp.pad(xt, ((1, 1), (1, 1), (0, 0)))


def _build_patches(x3, xc_ref, H, W, Cin):
    R = H * W
    for kh in range(3):
        for kw in range(3):
            t = kh * 3 + kw
            v = x3[kh:kh + H, kw:kw + W, :].reshape(R, Cin)
            xc_ref[:, t * Cin:(t + 1) * Cin] = v


def _stats_kernel(H, W, Cin, ipb, x_ref, w_ref, stats_ref, xc_ref):
    s_acc = ss_acc = None
    for j in range(ipb):
        _build_patches(x_ref[j], xc_ref, H, W, Cin)
        y = jnp.dot(xc_ref[...], w_ref[...], preferred_element_type=jnp.float32)
        s = jnp.sum(y, axis=0)
        ss = jnp.sum(y * y, axis=0)
        s_acc = s if s_acc is None else s_acc + s
        ss_acc = ss if ss_acc is None else ss_acc + ss
    stats_ref[0, 0, :] = s_acc
    stats_ref[0, 1, :] = ss_acc


def _out_kernel(H, W, Cin, ipb, x_ref, w_ref, scale_ref, shift_ref, o_ref, xc_ref):
    for j in range(ipb):
        _build_patches(x_ref[j], xc_ref, H, W, Cin)
        yt = jax.lax.dot_general(
            w_ref[...], xc_ref[...],
            dimension_numbers=(((0,), (1,)), ((), ())),
            preferred_element_type=jnp.float32)
        o_ref[j] = jnp.maximum(yt * scale_ref[...] + shift_ref[...], 0.0)


def kernel(x_nchw, w_oihw, gamma, beta):
    N, Cin, H, W = x_nchw.shape
    Cout = w_oihw.shape[0]
    K = 9 * Cin
    R = H * W
    ipb = _IPB
    while N % ipb:
        ipb //= 2
    G = N // ipb

    w_mat = jnp.transpose(w_oihw, (2, 3, 1, 0)).reshape(K, Cout).astype(jnp.bfloat16)

    params = pltpu.CompilerParams(
        dimension_semantics=("arbitrary",),
        vmem_limit_bytes=_VMEM_LIMIT)

    xp = pl.pallas_call(
        functools.partial(_prep_kernel, ipb),
        out_shape=jax.ShapeDtypeStruct((N, H + 2, W + 2, Cin), jnp.bfloat16),
        grid=(G,),
        in_specs=[pl.BlockSpec((ipb, Cin, H, W), lambda i: (i, 0, 0, 0))],
        out_specs=pl.BlockSpec((ipb, H + 2, W + 2, Cin), lambda i: (i, 0, 0, 0)),
        compiler_params=params,
    )(x_nchw)

    stats = pl.pallas_call(
        functools.partial(_stats_kernel, H, W, Cin, ipb),
        out_shape=jax.ShapeDtypeStruct((G, 2, Cout), jnp.float32),
        grid=(G,),
        in_specs=[
            pl.BlockSpec((ipb, H + 2, W + 2, Cin), lambda i: (i, 0, 0, 0)),
            pl.BlockSpec((K, Cout), lambda i: (0, 0)),
        ],
        out_specs=pl.BlockSpec((1, 2, Cout), lambda i: (i, 0, 0)),
        scratch_shapes=[pltpu.VMEM((R, K), jnp.bfloat16)],
        compiler_params=params,
    )(xp, w_mat)

    tot = jnp.sum(stats, axis=0)
    cnt = jnp.float32(N * R)
    mean = tot[0] / cnt
    var = tot[1] / cnt - mean * mean
    inv_std = jax.lax.rsqrt(var + _BN_EPS)
    scale = (gamma.astype(jnp.float32) * inv_std).reshape(Cout, 1)
    shift = (beta.astype(jnp.float32) - mean * gamma.astype(jnp.float32)
             * inv_std).reshape(Cout, 1)

    out_flat = pl.pallas_call(
        functools.partial(_out_kernel, H, W, Cin, ipb),
        out_shape=jax.ShapeDtypeStruct((N, Cout, R), jnp.float32),
        grid=(G,),
        in_specs=[
            pl.BlockSpec((ipb, H + 2, W + 2, Cin), lambda i: (i, 0, 0, 0)),
            pl.BlockSpec((K, Cout), lambda i: (0, 0)),
            pl.BlockSpec((Cout, 1), lambda i: (0, 0)),
            pl.BlockSpec((Cout, 1), lambda i: (0, 0)),
        ],
        out_specs=pl.BlockSpec((ipb, Cout, R), lambda i: (i, 0, 0)),
        scratch_shapes=[pltpu.VMEM((R, K), jnp.bfloat16)],
        compiler_params=params,
    )(xp, w_mat, scale, shift)

    return out_flat.reshape(N, Cout, H, W)


# speedup vs baseline: 1.9353x; 1.9353x over previous
import functools

import jax
import jax.numpy as jnp
from jax.experimental import pallas as pl
from jax.experimental.pallas import tpu as pltpu

_BN_EPS = 1e-5
_VMEM_LIMIT = 64 * 1024 * 1024
_IPB = 4


def _build_patches(x3, xc_ref, H, W, Cin):
    R = H * W
    for kh in range(3):
        for kw in range(3):
            t = kh * 3 + kw
            dh, dw = kh - 1, kw - 1
            r0, r1 = max(0, dh), min(H, H + dh)
            c0, c1 = max(0, dw), min(W, W + dw)
            v = x3[r0:r1, c0:c1, :]
            v = jnp.pad(v, ((r0 - dh, (H + dh) - r1),
                            (c0 - dw, (W + dw) - c1), (0, 0)))
            xc_ref[:, t * Cin:(t + 1) * Cin] = v.reshape(R, Cin)


def _stats_kernel(H, W, Cin, ipb, x_ref, w_ref, stats_ref, xc_ref):
    s_acc = ss_acc = None
    for j in range(ipb):
        _build_patches(x_ref[j].astype(jnp.bfloat16), xc_ref, H, W, Cin)
        y = jnp.dot(xc_ref[...], w_ref[...], preferred_element_type=jnp.float32)
        s = jnp.sum(y, axis=0)
        ss = jnp.sum(y * y, axis=0)
        s_acc = s if s_acc is None else s_acc + s
        ss_acc = ss if ss_acc is None else ss_acc + ss
    stats_ref[0, 0, :] = s_acc
    stats_ref[0, 1, :] = ss_acc


def _out_kernel(H, W, Cin, ipb, x_ref, w_ref, scale_ref, shift_ref, o_ref, xc_ref):
    for j in range(ipb):
        _build_patches(x_ref[j].astype(jnp.bfloat16), xc_ref, H, W, Cin)
        y = jnp.dot(xc_ref[...], w_ref[...], preferred_element_type=jnp.float32)
        o_ref[j] = jnp.maximum(y * scale_ref[...] + shift_ref[...], 0.0)


def kernel(x_nchw, w_oihw, gamma, beta):
    N, Cin, H, W = x_nchw.shape
    Cout = w_oihw.shape[0]
    K = 9 * Cin
    R = H * W
    ipb = _IPB
    while N % ipb:
        ipb //= 2
    G = N // ipb

    x_nhwc = jnp.transpose(x_nchw, (0, 2, 3, 1))
    w_mat = jnp.transpose(w_oihw, (2, 3, 1, 0)).reshape(K, Cout).astype(jnp.bfloat16)

    params = pltpu.CompilerParams(
        dimension_semantics=("arbitrary",),
        vmem_limit_bytes=_VMEM_LIMIT)

    stats = pl.pallas_call(
        functools.partial(_stats_kernel, H, W, Cin, ipb),
        out_shape=jax.ShapeDtypeStruct((G, 2, Cout), jnp.float32),
        grid=(G,),
        in_specs=[
            pl.BlockSpec((ipb, H, W, Cin), lambda i: (i, 0, 0, 0)),
            pl.BlockSpec((K, Cout), lambda i: (0, 0)),
        ],
        out_specs=pl.BlockSpec((1, 2, Cout), lambda i: (i, 0, 0)),
        scratch_shapes=[pltpu.VMEM((R, K), jnp.bfloat16)],
        compiler_params=params,
    )(x_nhwc, w_mat)

    tot = jnp.sum(stats, axis=0)
    cnt = jnp.float32(N * R)
    mean = tot[0] / cnt
    var = tot[1] / cnt - mean * mean
    inv_std = jax.lax.rsqrt(var + _BN_EPS)
    scale = (gamma.astype(jnp.float32) * inv_std).reshape(1, Cout)
    shift = (beta.astype(jnp.float32) - mean * gamma.astype(jnp.float32)
             * inv_std).reshape(1, Cout)

    out_rows = pl.pallas_call(
        functools.partial(_out_kernel, H, W, Cin, ipb),
        out_shape=jax.ShapeDtypeStruct((N, R, Cout), jnp.float32),
        grid=(G,),
        in_specs=[
            pl.BlockSpec((ipb, H, W, Cin), lambda i: (i, 0, 0, 0)),
            pl.BlockSpec((K, Cout), lambda i: (0, 0)),
            pl.BlockSpec((1, Cout), lambda i: (0, 0)),
            pl.BlockSpec((1, Cout), lambda i: (0, 0)),
        ],
        out_specs=pl.BlockSpec((ipb, R, Cout), lambda i: (i, 0, 0)),
        scratch_shapes=[pltpu.VMEM((R, K), jnp.bfloat16)],
        compiler_params=params,
    )(x_nhwc, w_mat, scale, shift)

    return jnp.transpose(out_rows.reshape(N, H, W, Cout), (0, 3, 1, 2))


# speedup vs baseline: 2.8706x; 1.4833x over previous
import functools

import jax
import jax.numpy as jnp
from jax.experimental import pallas as pl
from jax.experimental.pallas import tpu as pltpu

_BN_EPS = 1e-5
_VMEM_LIMIT = 64 * 1024 * 1024
_IPB = 4


def _build_patches(x3, xc_ref, H, W, Cin):
    R = H * W
    for kh in range(3):
        for kw in range(3):
            t = kh * 3 + kw
            dh, dw = kh - 1, kw - 1
            r0, r1 = max(0, dh), min(H, H + dh)
            c0, c1 = max(0, dw), min(W, W + dw)
            v = x3[r0:r1, c0:c1, :]
            v = jnp.pad(v, ((r0 - dh, (H + dh) - r1),
                            (c0 - dw, (W + dw) - c1), (0, 0)))
            xc_ref[:, t * Cin:(t + 1) * Cin] = v.reshape(R, Cin)


def _conv_kernel(H, W, Cin, ipb, x_ref, w_ref, y_ref, stats_ref, xc_ref):
    s_acc = ss_acc = None
    for j in range(ipb):
        _build_patches(x_ref[j].astype(jnp.bfloat16), xc_ref, H, W, Cin)
        y = jnp.dot(xc_ref[...], w_ref[...], preferred_element_type=jnp.float32)
        y_ref[j] = y.astype(jnp.bfloat16)
        s = jnp.sum(y, axis=0)
        ss = jnp.sum(y * y, axis=0)
        s_acc = s if s_acc is None else s_acc + s
        ss_acc = ss if ss_acc is None else ss_acc + ss
    stats_ref[0, 0, :] = s_acc
    stats_ref[0, 1, :] = ss_acc


def _bn_relu_kernel(cnt, ipb, y_ref, stats_ref, g_ref, b_ref, o_ref):
    tot = jnp.sum(stats_ref[...], axis=0)
    mean = tot[0:1] / cnt
    var = tot[1:2] / cnt - mean * mean
    inv_std = jax.lax.rsqrt(var + _BN_EPS)
    scale = g_ref[...] * inv_std
    shift = b_ref[...] - mean * scale
    for j in range(ipb):
        y = y_ref[j].astype(jnp.float32)
        o_ref[j] = jnp.maximum(y * scale + shift, 0.0)


def kernel(x_nchw, w_oihw, gamma, beta):
    N, Cin, H, W = x_nchw.shape
    Cout = w_oihw.shape[0]
    K = 9 * Cin
    R = H * W
    ipb = _IPB
    while N % ipb:
        ipb //= 2
    G = N // ipb

    x_nhwc = jnp.transpose(x_nchw, (0, 2, 3, 1))
    w_mat = jnp.transpose(w_oihw, (2, 3, 1, 0)).reshape(K, Cout).astype(jnp.bfloat16)

    params = pltpu.CompilerParams(
        dimension_semantics=("arbitrary",),
        vmem_limit_bytes=_VMEM_LIMIT)

    y_rows, stats = pl.pallas_call(
        functools.partial(_conv_kernel, H, W, Cin, ipb),
        out_shape=(jax.ShapeDtypeStruct((N, R, Cout), jnp.bfloat16),
                   jax.ShapeDtypeStruct((G, 2, Cout), jnp.float32)),
        grid=(G,),
        in_specs=[
            pl.BlockSpec((ipb, H, W, Cin), lambda i: (i, 0, 0, 0)),
            pl.BlockSpec((K, Cout), lambda i: (0, 0)),
        ],
        out_specs=(
            pl.BlockSpec((ipb, R, Cout), lambda i: (i, 0, 0)),
            pl.BlockSpec((1, 2, Cout), lambda i: (i, 0, 0)),
        ),
        scratch_shapes=[pltpu.VMEM((R, K), jnp.bfloat16)],
        compiler_params=params,
    )(x_nhwc, w_mat)

    out_rows = pl.pallas_call(
        functools.partial(_bn_relu_kernel, float(N * R), ipb),
        out_shape=jax.ShapeDtypeStruct((N, R, Cout), jnp.float32),
        grid=(G,),
        in_specs=[
            pl.BlockSpec((ipb, R, Cout), lambda i: (i, 0, 0)),
            pl.BlockSpec((G, 2, Cout), lambda i: (0, 0, 0)),
            pl.BlockSpec((1, Cout), lambda i: (0, 0)),
            pl.BlockSpec((1, Cout), lambda i: (0, 0)),
        ],
        out_specs=pl.BlockSpec((ipb, R, Cout), lambda i: (i, 0, 0)),
        compiler_params=params,
    )(y_rows, stats, gamma.reshape(1, Cout).astype(jnp.float32),
      beta.reshape(1, Cout).astype(jnp.float32))

    return jnp.transpose(out_rows.reshape(N, H, W, Cout), (0, 3, 1, 2))


# speedup vs baseline: 2.9598x; 1.0311x over previous
import functools

import jax
import jax.numpy as jnp
from jax.experimental import pallas as pl
from jax.experimental.pallas import tpu as pltpu

_BN_EPS = 1e-5
_VMEM_LIMIT = 64 * 1024 * 1024
_IPB = 8


def _build_patches(x3, xc_ref, H, W, Cin):
    R = H * W
    for kh in range(3):
        for kw in range(3):
            t = kh * 3 + kw
            dh, dw = kh - 1, kw - 1
            r0, r1 = max(0, dh), min(H, H + dh)
            c0, c1 = max(0, dw), min(W, W + dw)
            v = x3[r0:r1, c0:c1, :]
            v = jnp.pad(v, ((r0 - dh, (H + dh) - r1),
                            (c0 - dw, (W + dw) - c1), (0, 0)))
            xc_ref[:, t * Cin:(t + 1) * Cin] = v.reshape(R, Cin)


def _conv_kernel(H, W, Cin, ipb, x_ref, w_ref, y_ref, stats_ref, xc_ref):
    s_acc = ss_acc = None
    for j in range(ipb):
        _build_patches(x_ref[j].astype(jnp.bfloat16), xc_ref, H, W, Cin)
        y = jnp.dot(xc_ref[...], w_ref[...], preferred_element_type=jnp.float32)
        y_ref[j] = y.astype(jnp.bfloat16)
        s = jnp.sum(y, axis=0)
        ss = jnp.sum(y * y, axis=0)
        s_acc = s if s_acc is None else s_acc + s
        ss_acc = ss if ss_acc is None else ss_acc + ss
    stats_ref[0, 0, :] = s_acc
    stats_ref[0, 1, :] = ss_acc


def _bn_relu_kernel(cnt, ipb, y_ref, stats_ref, g_ref, b_ref, o_ref):
    tot = jnp.sum(stats_ref[...], axis=0)
    mean = tot[0:1] / cnt
    var = tot[1:2] / cnt - mean * mean
    inv_std = jax.lax.rsqrt(var + _BN_EPS)
    scale = g_ref[...] * inv_std
    shift = b_ref[...] - mean * scale
    for j in range(ipb):
        y = y_ref[j].astype(jnp.float32)
        o_ref[j] = jnp.maximum(y * scale + shift, 0.0)


def kernel(x_nchw, w_oihw, gamma, beta):
    N, Cin, H, W = x_nchw.shape
    Cout = w_oihw.shape[0]
    K = 9 * Cin
    R = H * W
    ipb = _IPB
    while N % ipb:
        ipb //= 2
    G = N // ipb

    x_nhwc = jnp.transpose(x_nchw, (0, 2, 3, 1))
    w_mat = jnp.transpose(w_oihw, (2, 3, 1, 0)).reshape(K, Cout).astype(jnp.bfloat16)

    params = pltpu.CompilerParams(
        dimension_semantics=("arbitrary",),
        vmem_limit_bytes=_VMEM_LIMIT)

    y_rows, stats = pl.pallas_call(
        functools.partial(_conv_kernel, H, W, Cin, ipb),
        out_shape=(jax.ShapeDtypeStruct((N, R, Cout), jnp.bfloat16),
                   jax.ShapeDtypeStruct((G, 2, Cout), jnp.float32)),
        grid=(G,),
        in_specs=[
            pl.BlockSpec((ipb, H, W, Cin), lambda i: (i, 0, 0, 0)),
            pl.BlockSpec((K, Cout), lambda i: (0, 0)),
        ],
        out_specs=(
            pl.BlockSpec((ipb, R, Cout), lambda i: (i, 0, 0)),
            pl.BlockSpec((1, 2, Cout), lambda i: (i, 0, 0)),
        ),
        scratch_shapes=[pltpu.VMEM((R, K), jnp.bfloat16)],
        compiler_params=params,
    )(x_nhwc, w_mat)

    out_rows = pl.pallas_call(
        functools.partial(_bn_relu_kernel, float(N * R), ipb),
        out_shape=jax.ShapeDtypeStruct((N, R, Cout), jnp.float32),
        grid=(G,),
        in_specs=[
            pl.BlockSpec((ipb, R, Cout), lambda i: (i, 0, 0)),
            pl.BlockSpec((G, 2, Cout), lambda i: (0, 0, 0)),
            pl.BlockSpec((1, Cout), lambda i: (0, 0)),
            pl.BlockSpec((1, Cout), lambda i: (0, 0)),
        ],
        out_specs=pl.BlockSpec((ipb, R, Cout), lambda i: (i, 0, 0)),
        compiler_params=params,
    )(y_rows, stats, gamma.reshape(1, Cout).astype(jnp.float32),
      beta.reshape(1, Cout).astype(jnp.float32))

    return jnp.transpose(out_rows.reshape(N, H, W, Cout), (0, 3, 1, 2))
